# Initial kernel scaffold; baseline (speedup 1.0000x reference)
import jax
import jax.numpy as jnp
from jax.experimental import pallas as pl


def kernel(x):
    raise NotImplementedError("write your pallas kernel here")


# speedup vs baseline: 25.4985x; 25.4985x over previous
import jax
import jax.numpy as jnp
from jax import lax
from jax.experimental import pallas as pl

_K = 2048
_ROWS_PER_BLOCK = 16


def _topk_mask_body(x_ref, o_ref):
    x = x_ref[...]
    u = lax.bitcast_convert_type(x, jnp.uint32)
    flip = jnp.where(u >> 31 != 0, jnp.uint32(0xFFFFFFFF), jnp.uint32(0x80000000))
    m = u ^ flip

    def body(i, prefix):
        bit = jnp.uint32(31) - i.astype(jnp.uint32)
        cand = prefix | (jnp.uint32(1) << bit)
        cnt = jnp.sum((m >= cand).astype(jnp.int32), axis=1, keepdims=True)
        return jnp.where(cnt >= _K, cand, prefix)

    prefix = jnp.zeros((x.shape[0], 1), dtype=jnp.uint32)
    thresh = lax.fori_loop(0, 32, body, prefix)
    keep = m >= thresh
    o_ref[...] = jnp.where(keep, jnp.maximum(x, 0.0), 0.0)


def kernel(x):
    rows, cols = x.shape
    grid = (rows // _ROWS_PER_BLOCK,)
    return pl.pallas_call(
        _topk_mask_body,
        grid=grid,
        in_specs=[pl.BlockSpec((_ROWS_PER_BLOCK, cols), lambda i: (i, 0))],
        out_specs=pl.BlockSpec((_ROWS_PER_BLOCK, cols), lambda i: (i, 0)),
        out_shape=jax.ShapeDtypeStruct((rows, cols), x.dtype),
    )(x)
